# baseline (speedup 1.0000x reference)
import functools

import jax
import jax.numpy as jnp
from jax import lax
from jax.experimental import pallas as pl
from jax.experimental.pallas import tpu as pltpu
from jax.experimental.pallas import tpu_sc as plsc

VOCAB_N = 1_000_000
D = 64
BATCH_N = 16384
HIST_N = 200
B_TOTAL = BATCH_N * HIST_N

NW = 32
PER_W = B_TOTAL // NW
SUB = 256
NSUB = PER_W // SUB
NPAIR = NSUB // 2
CHUNK = 128
LANES = 16
NCREG = D // LANES

XROWS_W = PER_W // CHUNK
OROWS_SUB = SUB // 2
OROWS_W = PER_W // 2

_MEAN_BLK = 8000
_MEAN_GRID = VOCAB_N // _MEAN_BLK


def _mean_body(t_ref, o_ref):
    i = pl.program_id(0)

    @pl.when(i == 0)
    def _():
        o_ref[...] = jnp.zeros_like(o_ref)

    s = jnp.sum(t_ref[...], axis=0, keepdims=True)
    o_ref[...] += jnp.broadcast_to(jnp.concatenate([s, s], axis=1), (8, 2 * D))

    @pl.when(i == _MEAN_GRID - 1)
    def _():
        o_ref[...] = o_ref[...] * (1.0 / VOCAB_N)


def _tc_mean(table):
    return pl.pallas_call(
        _mean_body,
        grid=(_MEAN_GRID,),
        in_specs=[pl.BlockSpec((_MEAN_BLK, D), lambda i: (i, 0))],
        out_specs=pl.BlockSpec((8, 2 * D), lambda i: (0, 0)),
        out_shape=jax.ShapeDtypeStruct((8, 2 * D), jnp.float32),
    )(table)


_mesh = plsc.VectorSubcoreMesh(core_axis_name="c", subcore_axis_name="s")


@functools.partial(
    pl.kernel,
    mesh=_mesh,
    compiler_params=pltpu.CompilerParams(use_tc_tiling_on_sc=True),
    out_type=jax.ShapeDtypeStruct((B_TOTAL // 2, 2 * D), jnp.float32),
    scratch_types=[
        pltpu.VMEM((2, 2, CHUNK), jnp.int32),
        pltpu.VMEM((2, 2, CHUNK), jnp.int32),
        pltpu.VMEM((2, SUB, 2 * D), jnp.float32),
        pltpu.VMEM((2, OROWS_SUB, 2 * D), jnp.float32),
        pltpu.VMEM((8, 2 * D), jnp.float32),
        pltpu.SemaphoreType.DMA,
        pltpu.SemaphoreType.DMA,
    ],
)
def _sc_gather_sub(x_hbm, table_hbm, center_hbm, out_hbm,
                   idx_v, q_v, rows_v, out_v, center_v, sem_g, sem_s):
    wid = lax.axis_index("s") * 2 + lax.axis_index("c")
    xbase = wid * XROWS_W
    obase = wid * OROWS_W

    pltpu.sync_copy(center_hbm, center_v)
    cregs = [center_v[0, pl.ds(LANES * c, LANES)] for c in range(NCREG)]

    def stage_and_fire(buf, s):
        pltpu.sync_copy(x_hbm.at[pl.ds(xbase + 2 * s, 2)], idx_v.at[buf])
        for t in range(2):
            for c in range(8):
                sl = pl.ds(LANES * c, LANES)
                q_v[buf, t, sl] = lax.shift_right_logical(idx_v[buf, t, sl], 1)
        for t in range(2):
            pltpu.async_copy(
                table_hbm.at[q_v.at[buf, t]],
                rows_v.at[buf, pl.ds(t * CHUNK, CHUNK)],
                sem_g,
            )

    def wait_gather(buf):
        for t in range(2):
            pltpu.make_async_copy(
                table_hbm.at[q_v.at[buf, t]],
                rows_v.at[buf, pl.ds(t * CHUNK, CHUNK)],
                sem_g,
            ).wait()

    def fire_scatter(buf, s):
        pltpu.async_copy(
            out_v.at[buf],
            out_hbm.at[pl.ds(obase + s * OROWS_SUB, OROWS_SUB)],
            sem_s,
        )

    def wait_scatter(buf, s):
        pltpu.make_async_copy(
            out_v.at[buf],
            out_hbm.at[pl.ds(obase + s * OROWS_SUB, OROWS_SUB)],
            sem_s,
        ).wait()

    def process(buf):
        def blk(bb, carry):
            t = lax.shift_right_logical(bb, 3)
            lane0 = LANES * lax.rem(bb, 8)
            pv = idx_v[buf, t, pl.ds(lane0, LANES)]
            for i in range(LANES):
                off = (pv[i] & 1) * D
                j = LANES * bb + i
                m = 8 * bb + (i // 2)
                halfcol = (i & 1) * D
                for c in range(NCREG):
                    src = rows_v[buf, j, pl.ds(off + LANES * c, LANES)]
                    out_v[buf, m, pl.ds(halfcol + LANES * c, LANES)] = (
                        src - cregs[c])
            return carry

        lax.fori_loop(0, SUB // LANES, blk, 0)

    stage_and_fire(0, 0)

    def body(k, carry):
        s0 = 2 * k
        s1 = s0 + 1
        @pl.when(k > 0)
        def _():
            wait_scatter(1, s0 - 1)
        stage_and_fire(1, s1)
        wait_gather(0)
        process(0)
        fire_scatter(0, s0)
        @pl.when(k < NPAIR - 1)
        def _():
            wait_scatter(0, s0)
            stage_and_fire(0, s0 + 2)
        wait_gather(1)
        process(1)
        fire_scatter(1, s1)
        return carry

    lax.fori_loop(0, NPAIR, body, 0)
    wait_scatter(0, NSUB - 2)
    wait_scatter(1, NSUB - 1)


def kernel(x, table):
    center = _tc_mean(table)
    x2 = x.reshape(-1).astype(jnp.int32).reshape(B_TOTAL // CHUNK, CHUNK)
    table2 = table.reshape(VOCAB_N // 2, 2 * D)
    out = _sc_gather_sub(x2, table2, center)
    return out.reshape(BATCH_N, HIST_N, D)
